# Initial kernel scaffold; baseline (speedup 1.0000x reference)
import jax
import jax.numpy as jnp
from jax.experimental import pallas as pl


def kernel(features, positions, W1, b1, W2, b2, grid_size):
    raise NotImplementedError("write your pallas kernel here")


# speedup vs baseline: 8.1686x; 8.1686x over previous
import functools

import jax
import jax.numpy as jnp
from jax import lax
from jax.experimental import pallas as pl
from jax.experimental.pallas import tpu as pltpu
from jax.experimental.pallas import tpu_sc as plsc

_NUM_CORES = 2
_NUM_SUBCORES = 16
_LANES = 16
_NW = _NUM_CORES * _NUM_SUBCORES


def _mlp_body(x_ref, w1_ref, b1_ref, w2_ref, b2_ref, o_ref):
    x = x_ref[...]
    h = jnp.dot(x, w1_ref[...], preferred_element_type=jnp.float32) + b1_ref[...]
    h = h * jax.nn.sigmoid(h)
    o_ref[...] = jnp.dot(h, w2_ref[...], preferred_element_type=jnp.float32) + b2_ref[...]


def _mlp(x, W1, b1, W2, b2, blk):
    R, D = x.shape
    grid = (R // blk,)
    return pl.pallas_call(
        _mlp_body,
        grid=grid,
        in_specs=[
            pl.BlockSpec((blk, D), lambda i: (i, 0)),
            pl.BlockSpec((D, D), lambda i: (0, 0)),
            pl.BlockSpec((1, D), lambda i: (0, 0)),
            pl.BlockSpec((D, D), lambda i: (0, 0)),
            pl.BlockSpec((1, D), lambda i: (0, 0)),
        ],
        out_specs=pl.BlockSpec((blk, D), lambda i: (i, 0)),
        out_shape=jax.ShapeDtypeStruct((R, D), jnp.float32),
    )(x, W1, b1, W2, b2)


def _make_gather(BM, D, B, M, N, chunk):
    qpw = BM // _NW
    nchunks = qpw // chunk
    mesh = plsc.VectorSubcoreMesh(core_axis_name="c", subcore_axis_name="s")

    @functools.partial(
        pl.kernel,
        mesh=mesh,
        out_type=jax.ShapeDtypeStruct((BM, D), jnp.float32),
        scratch_types=[
            pltpu.VMEM((qpw,), jnp.float32),
            pltpu.VMEM((qpw,), jnp.float32),
            pltpu.VMEM((qpw,), jnp.float32),
            pltpu.VMEM((_LANES,), jnp.float32),
            pltpu.VMEM((nchunks, chunk), jnp.int32),
            pltpu.VMEM((chunk, D), jnp.float32),
            pltpu.SemaphoreType.DMA,
        ],
    )
    def gather(px_hbm, py_hbm, pz_hbm, gs_hbm, table_hbm, out_hbm,
               px_v, py_v, pz_v, gs_v, idx_v, rows_v, sem):
        wid = lax.axis_index("s") * _NUM_CORES + lax.axis_index("c")
        base = wid * qpw
        boff = (base // M) * N

        pltpu.sync_copy(px_hbm.at[pl.ds(base, qpw)], px_v)
        pltpu.sync_copy(py_hbm.at[pl.ds(base, qpw)], py_v)
        pltpu.sync_copy(pz_hbm.at[pl.ds(base, qpw)], pz_v)
        pltpu.sync_copy(gs_hbm, gs_v)

        gsf = gs_v[...]
        gsi = gsf.astype(jnp.int32)

        def compute_group(g, carry):
            off = g * _LANES
            xi = ((px_v[pl.ds(off, _LANES)] + 0.5) * gsf).astype(jnp.int32)
            yi = ((py_v[pl.ds(off, _LANES)] + 0.5) * gsf).astype(jnp.int32)
            zi = ((pz_v[pl.ds(off, _LANES)] + 0.5) * gsf).astype(jnp.int32)
            flat = (xi * gsi + yi) * gsi + zi
            flat = jnp.clip(flat, 0, N - 1) + boff
            idx_v[g // (chunk // _LANES),
                  pl.ds((g % (chunk // _LANES)) * _LANES, _LANES)] = flat
            return carry

        lax.fori_loop(0, qpw // _LANES, compute_group, 0)

        def do_chunk(i, carry):
            pltpu.async_copy(table_hbm.at[idx_v.at[i]], rows_v, sem).wait()
            pltpu.sync_copy(rows_v, out_hbm.at[pl.ds(base + i * chunk, chunk)])
            return carry

        lax.fori_loop(0, nchunks, do_chunk, 0)

    return gather


def kernel(features, positions, W1, b1, W2, b2, grid_size):
    B, N, D = features.shape
    M = positions.shape[1]
    BM = B * M

    table = _mlp(features.reshape(B * N, D), W1, b1.reshape(1, D),
                 W2, b2.reshape(1, D), blk=512)

    pos_flat = positions.reshape(BM, 3)
    px = pos_flat[:, 0].reshape(BM)
    py = pos_flat[:, 1].reshape(BM)
    pz = pos_flat[:, 2].reshape(BM)
    gs = jnp.full((_LANES,), grid_size, jnp.float32)

    out = _make_gather(BM, D, B, M, N, chunk=128)(px, py, pz, gs, table)
    return out.reshape(B, M, D)
